# Initial kernel scaffold; baseline (speedup 1.0000x reference)
import jax
import jax.numpy as jnp
from jax.experimental import pallas as pl


def kernel(x, table, W, b):
    raise NotImplementedError("write your pallas kernel here")


# speedup vs baseline: 51.9124x; 51.9124x over previous
import functools

import jax
import jax.numpy as jnp
from jax import lax
from jax.experimental import pallas as pl
from jax.experimental.pallas import tpu as pltpu
from jax.experimental.pallas import tpu_sc as plsc

_VOCAB = 10001
_EMB = 100
_NCLS = 2
_B = 4096
_L = 200

_VP = 10016
_BSLOT = 10008
_LPAD = 208

_info = plsc.get_sparse_core_info()
_NC, _NS = _info.num_cores, _info.num_subcores
_NW = _NC * _NS
_ROWS = _B // _NW
_GROUPS = _ROWS // 16


def _p_body(tab_ref, w_ref, b_ref, p_ref):
    w = w_ref[...] * (1.0 / _L)
    p = lax.dot_general(w, tab_ref[...], (((1,), (1,)), ((), ())),
                        preferred_element_type=jnp.float32)
    col = lax.broadcasted_iota(jnp.int32, (_NCLS, _VP), 1)
    p = jnp.where(col == 0, 0.0, p)
    p = jnp.where(col == _BSLOT, b_ref[...], p)
    p_ref[...] = p


def _sc_body(x_hbm, p_hbm, out_hbm, xbuf, p0, p1, obuf):
    wid = lax.axis_index("s") * _NC + lax.axis_index("c")
    pltpu.sync_copy(x_hbm.at[pl.ds(wid * _ROWS * _LPAD, _ROWS * _LPAD)], xbuf)
    pltpu.sync_copy(p_hbm.at[0], p0)
    pltpu.sync_copy(p_hbm.at[1], p1)

    def group(g, carry):
        rloc = g * 16 + lax.iota(jnp.int32, 16)
        ridx = rloc * _LPAD
        acc0 = jnp.zeros((16,), jnp.float32)
        acc1 = jnp.zeros((16,), jnp.float32)
        for j in range(_LPAD):
            xv = plsc.load_gather(xbuf, [ridx + j])
            acc0 = acc0 + plsc.load_gather(p0, [xv])
            acc1 = acc1 + plsc.load_gather(p1, [xv])
        plsc.store_scatter(obuf, [rloc * _NCLS], acc0)
        plsc.store_scatter(obuf, [rloc * _NCLS + 1], acc1)
        return carry

    lax.fori_loop(0, _GROUPS, group, 0)
    pltpu.sync_copy(obuf, out_hbm.at[pl.ds(wid * _ROWS * _NCLS, _ROWS * _NCLS)])


def kernel(x, table, W, b):
    tpad = jnp.zeros((_VP, _EMB), jnp.float32).at[:_VOCAB].set(table)
    xpad = jnp.concatenate(
        [x,
         jnp.full((_B, 1), _BSLOT, jnp.int32),
         jnp.zeros((_B, _LPAD - _L - 1), jnp.int32)], axis=1).reshape(-1)
    bcol = b.reshape(_NCLS, 1)

    p = pl.pallas_call(
        _p_body,
        out_shape=jax.ShapeDtypeStruct((_NCLS, _VP), jnp.float32),
    )(tpad, W, bcol)

    mesh = plsc.VectorSubcoreMesh(core_axis_name="c", subcore_axis_name="s")
    sc = functools.partial(
        pl.kernel,
        mesh=mesh,
        out_type=jax.ShapeDtypeStruct((_B * _NCLS,), jnp.float32),
        scratch_types=[
            pltpu.VMEM((_ROWS * _LPAD,), jnp.int32),
            pltpu.VMEM((_VP,), jnp.float32),
            pltpu.VMEM((_VP,), jnp.float32),
            pltpu.VMEM((_ROWS * _NCLS,), jnp.float32),
        ],
        compiler_params=pltpu.CompilerParams(
            needs_layout_passes=False, use_tc_tiling_on_sc=False),
    )(_sc_body)
    return sc(xpad, p).reshape(_B, _NCLS)


# speedup vs baseline: 55.5930x; 1.0709x over previous
import functools

import jax
import jax.numpy as jnp
from jax import lax
from jax.experimental import pallas as pl
from jax.experimental.pallas import tpu as pltpu
from jax.experimental.pallas import tpu_sc as plsc

_VOCAB = 10001
_EMB = 100
_NCLS = 2
_B = 4096
_L = 200

_VP = 10016
_BSLOT = 10008

_info = plsc.get_sparse_core_info()
_NC, _NS = _info.num_cores, _info.num_subcores
_NW = _NC * _NS
_ROWS = _B // _NW
_GROUPS = _ROWS // 16


def _p_body(tab_ref, w_ref, b_ref, p_ref):
    w = w_ref[...] * (1.0 / _L)
    p = lax.dot_general(w, tab_ref[...], (((1,), (1,)), ((), ())),
                        preferred_element_type=jnp.float32)
    p = jnp.concatenate([p, jnp.zeros((_NCLS, _VP - _VOCAB), jnp.float32)],
                        axis=1)
    col = lax.broadcasted_iota(jnp.int32, (_NCLS, _VP), 1)
    p = jnp.where(col == 0, 0.0, p)
    p = jnp.where(col == _BSLOT, b_ref[...], p)
    p_ref[...] = p


def _sc_body(x_hbm, p_hbm, out_hbm, xbuf, p0, p1, obuf):
    wid = lax.axis_index("s") * _NC + lax.axis_index("c")
    pltpu.sync_copy(x_hbm.at[pl.ds(wid * _ROWS * _L, _ROWS * _L)], xbuf)
    pltpu.sync_copy(p_hbm.at[0], p0)
    pltpu.sync_copy(p_hbm.at[1], p1)

    def group(g, carry):
        rloc = g * 16 + lax.iota(jnp.int32, 16)
        bslot = jnp.full((16,), _BSLOT, jnp.int32)
        acc0 = plsc.load_gather(p0, [bslot])
        acc1 = plsc.load_gather(p1, [bslot])
        ridx = rloc * _L
        for j in range(_L):
            xv = plsc.load_gather(xbuf, [ridx + j])
            acc0 = acc0 + plsc.load_gather(p0, [xv])
            acc1 = acc1 + plsc.load_gather(p1, [xv])
        plsc.store_scatter(obuf, [rloc * _NCLS], acc0)
        plsc.store_scatter(obuf, [rloc * _NCLS + 1], acc1)
        return carry

    lax.fori_loop(0, _GROUPS, group, 0)
    pltpu.sync_copy(obuf, out_hbm.at[pl.ds(wid * _ROWS * _NCLS, _ROWS * _NCLS)])


def kernel(x, table, W, b):
    p = pl.pallas_call(
        _p_body,
        out_shape=jax.ShapeDtypeStruct((_NCLS, _VP), jnp.float32),
    )(table, W, b.reshape(_NCLS, 1))

    mesh = plsc.VectorSubcoreMesh(core_axis_name="c", subcore_axis_name="s")
    sc = functools.partial(
        pl.kernel,
        mesh=mesh,
        out_type=jax.ShapeDtypeStruct((_B * _NCLS,), jnp.float32),
        scratch_types=[
            pltpu.VMEM((_ROWS * _L,), jnp.int32),
            pltpu.VMEM((_VP,), jnp.float32),
            pltpu.VMEM((_VP,), jnp.float32),
            pltpu.VMEM((_ROWS * _NCLS,), jnp.float32),
        ],
        compiler_params=pltpu.CompilerParams(
            needs_layout_passes=False, use_tc_tiling_on_sc=False),
    )(_sc_body)
    return sc(x.reshape(-1), p).reshape(_B, _NCLS)


# speedup vs baseline: 76.6524x; 1.3788x over previous
import functools

import jax
import jax.numpy as jnp
from jax import lax
from jax.experimental import pallas as pl
from jax.experimental.pallas import tpu as pltpu
from jax.experimental.pallas import tpu_sc as plsc

_VOCAB = 10001
_EMB = 100
_NCLS = 2
_B = 4096
_L = 200

_VP = 10016
_BSLOT = 10008

_info = plsc.get_sparse_core_info()
_NC, _NS = _info.num_cores, _info.num_subcores
_NW = _NC * _NS
_ROWS = _B // _NW
_GROUPS = _ROWS // 16
_JUNROLL = 10


def _round_bf16_bits(u):
    return (u + 0x7FFF + ((u >> 16) & 1)) & jnp.uint32(0xFFFF0000)


def _p_body(tab_ref, w_ref, b_ref, p_ref):
    w = w_ref[...] * (1.0 / _L)
    p = lax.dot_general(w, tab_ref[...], (((1,), (1,)), ((), ())),
                        preferred_element_type=jnp.float32)
    p = jnp.concatenate([p, jnp.zeros((_NCLS, _VP - _VOCAB), jnp.float32)],
                        axis=1)
    col = lax.broadcasted_iota(jnp.int32, (_NCLS, _VP), 1)
    p = jnp.where(col == 0, 0.0, p)
    p = jnp.where(col == _BSLOT, b_ref[...], p)
    u = lax.bitcast_convert_type(p, jnp.uint32)
    hi = _round_bf16_bits(u[0:1, :])
    lo = _round_bf16_bits(u[1:2, :]) >> 16
    p_ref[...] = lax.bitcast_convert_type(hi | lo, jnp.int32)


def _sc_body(xt_hbm, p_hbm, out_hbm, xbuf, pp, obuf):
    wid = lax.axis_index("s") * _NC + lax.axis_index("c")
    base = wid * _ROWS
    pltpu.sync_copy(xt_hbm.at[:, pl.ds(base, _ROWS)], xbuf)
    pltpu.sync_copy(p_hbm.at[0], pp)

    hi_mask = jnp.full((16,), -65536, jnp.int32)

    def group(g, carry0):
        rloc = g * 16 + lax.iota(jnp.int32, 16)
        bv = plsc.bitcast(
            plsc.load_gather(pp, [jnp.full((16,), _BSLOT, jnp.int32)]),
            jnp.int32)
        acc0 = plsc.bitcast(bv & hi_mask, jnp.float32)
        acc1 = plsc.bitcast(bv << 16, jnp.float32)

        def chunk(jc, carry):
            a0, a1 = carry
            for k in range(_JUNROLL):
                j = jc * _JUNROLL + k
                xv = plsc.load_gather(xbuf, [jnp.full((16,), j, jnp.int32),
                                             rloc])
                pk = plsc.load_gather(pp, [xv])
                a0 = a0 + plsc.bitcast(pk & hi_mask, jnp.float32)
                a1 = a1 + plsc.bitcast(pk << 16, jnp.float32)
            return a0, a1

        acc0, acc1 = lax.fori_loop(0, _L // _JUNROLL, chunk, (acc0, acc1))
        plsc.store_scatter(obuf, [rloc * _NCLS], acc0)
        plsc.store_scatter(obuf, [rloc * _NCLS + 1], acc1)
        return carry0

    lax.fori_loop(0, _GROUPS, group, 0)
    pltpu.sync_copy(obuf, out_hbm.at[pl.ds(base * _NCLS, _ROWS * _NCLS)])


def kernel(x, table, W, b):
    p = pl.pallas_call(
        _p_body,
        out_shape=jax.ShapeDtypeStruct((1, _VP), jnp.int32),
    )(table, W, b.reshape(_NCLS, 1))

    mesh = plsc.VectorSubcoreMesh(core_axis_name="c", subcore_axis_name="s")
    sc = functools.partial(
        pl.kernel,
        mesh=mesh,
        out_type=jax.ShapeDtypeStruct((_B * _NCLS,), jnp.float32),
        scratch_types=[
            pltpu.VMEM((_L, _ROWS), jnp.int32),
            pltpu.VMEM((_VP,), jnp.int32),
            pltpu.VMEM((_ROWS * _NCLS,), jnp.float32),
        ],
        compiler_params=pltpu.CompilerParams(
            needs_layout_passes=False, use_tc_tiling_on_sc=False),
    )(_sc_body)
    return sc(x.T, p).reshape(_B, _NCLS)


# speedup vs baseline: 77.8175x; 1.0152x over previous
import functools

import jax
import jax.numpy as jnp
from jax import lax
from jax.experimental import pallas as pl
from jax.experimental.pallas import tpu as pltpu
from jax.experimental.pallas import tpu_sc as plsc

_VOCAB = 10001
_EMB = 100
_NCLS = 2
_B = 4096
_L = 200

_VP = 10016
_BSLOT = 10008

_info = plsc.get_sparse_core_info()
_NC, _NS = _info.num_cores, _info.num_subcores
_NW = _NC * _NS
_ROWS = _B // _NW
_GROUPS = _ROWS // 16
_JUNROLL = 10


def _round_bf16_bits(u):
    return (u + 0x7FFF + ((u >> 16) & 1)) & jnp.uint32(0xFFFF0000)


def _p_body(tab_ref, w_ref, b_ref, p_ref):
    w = w_ref[...] * (1.0 / _L)
    p = lax.dot_general(w, tab_ref[...], (((1,), (1,)), ((), ())),
                        preferred_element_type=jnp.float32)
    p = jnp.concatenate([p, jnp.zeros((_NCLS, _VP - _VOCAB), jnp.float32)],
                        axis=1)
    col = lax.broadcasted_iota(jnp.int32, (_NCLS, _VP), 1)
    p = jnp.where(col == 0, 0.0, p)
    p = jnp.where(col == _BSLOT, b_ref[...], p)
    u = lax.bitcast_convert_type(p, jnp.uint32)
    hi = _round_bf16_bits(u[0:1, :])
    lo = _round_bf16_bits(u[1:2, :]) >> 16
    p_ref[...] = lax.bitcast_convert_type(hi | lo, jnp.int32)


def _sc_body(xt_hbm, p_hbm, out_hbm, xbuf, pp, obuf):
    wid = lax.axis_index("s") * _NC + lax.axis_index("c")
    base = wid * _ROWS
    pltpu.sync_copy(xt_hbm.at[:, pl.ds(base, _ROWS)], xbuf)
    pltpu.sync_copy(p_hbm.at[0], pp)

    hi_mask = jnp.full((16,), -65536, jnp.int32)

    def group(g, carry0):
        rloc = g * 16 + lax.iota(jnp.int32, 16)
        bv = plsc.bitcast(
            plsc.load_gather(pp, [jnp.full((16,), _BSLOT, jnp.int32)]),
            jnp.int32)
        acc0 = plsc.bitcast(bv & hi_mask, jnp.float32)
        acc1 = plsc.bitcast(bv << 16, jnp.float32)

        def chunk(jc, carry):
            a0, a1 = carry
            for k in range(_JUNROLL):
                j = jc * _JUNROLL + k
                xv = plsc.load_gather(xbuf, [jnp.full((16,), j, jnp.int32),
                                             rloc])
                pk = plsc.load_gather(pp, [xv])
                a0 = a0 + plsc.bitcast(pk & hi_mask, jnp.float32)
                a1 = a1 + plsc.bitcast(pk << 16, jnp.float32)
            return a0, a1

        acc0, acc1 = lax.fori_loop(0, _L // _JUNROLL, chunk, (acc0, acc1))
        zv = jnp.zeros((16,), jnp.int32)
        plsc.store_scatter(obuf, [rloc, zv], acc0)
        plsc.store_scatter(obuf, [rloc, zv + 1], acc1)
        return carry0

    lax.fori_loop(0, _GROUPS, group, 0)
    pltpu.sync_copy(obuf, out_hbm.at[pl.ds(base, _ROWS)])


def kernel(x, table, W, b):
    p = pl.pallas_call(
        _p_body,
        out_shape=jax.ShapeDtypeStruct((1, _VP), jnp.int32),
    )(table, W, b.reshape(_NCLS, 1))

    mesh = plsc.VectorSubcoreMesh(core_axis_name="c", subcore_axis_name="s")
    sc = functools.partial(
        pl.kernel,
        mesh=mesh,
        out_type=jax.ShapeDtypeStruct((_B, _NCLS), jnp.float32),
        scratch_types=[
            pltpu.VMEM((_L, _ROWS), jnp.int32),
            pltpu.VMEM((_VP,), jnp.int32),
            pltpu.VMEM((_ROWS, _NCLS), jnp.float32),
        ],
        compiler_params=pltpu.CompilerParams(
            needs_layout_passes=False, use_tc_tiling_on_sc=False),
    )(_sc_body)
    return sc(x.T, p)
